# baseline (speedup 1.0000x reference)
import jax
import jax.numpy as jnp
from jax import lax
from jax.experimental import pallas as pl
from jax.experimental.pallas import tpu as pltpu
from jax.experimental.pallas import tpu_sc as plsc

N = 100000
D_X = 128
PE_DIM = 8
HID = 128
OUT = 16
G = 512

NW = 32
NOCT = N // 8
BASE_O = NOCT // NW
EXTRA_O = NOCT % NW
CHUNK = 192
NFULL = 16

BLK = 2000
NB = N // BLK


def _sc_segsum_body(x_hbm, b_hbm, out_hbm, outc_hbm, rb0, rb1, ib0, ib1,
                    acc, accc, sem0, sem1):
    wid = lax.axis_index("c") * 16 + lax.axis_index("s")
    start_o = wid * BASE_O + jnp.minimum(wid, EXTRA_O)
    count_o = BASE_O + (wid < EXTRA_O).astype(jnp.int32)
    row0 = pl.multiple_of(start_o * 8, 8)
    rem_o = count_o - NFULL * (CHUNK // 8)

    @pl.loop(0, G)
    def _zero(r):
        for k in range(D_X // 16):
            acc[r, pl.ds(16 * k, 16)] = jnp.zeros((16,), jnp.float32)
        accc[pl.ds(r * 16, 16)] = jnp.zeros((16,), jnp.float32)

    def _start(c, rb, ib, sem):
        s = pl.multiple_of(row0 + c * CHUNK, 8)
        pltpu.async_copy(x_hbm.at[pl.ds(s, CHUNK)], rb, sem)
        pltpu.async_copy(b_hbm.at[pl.ds(s, CHUNK)],
                         ib.at[pl.ds(0, CHUNK)], sem)

    def _wait(c, rb, ib, sem):
        s = pl.multiple_of(row0 + c * CHUNK, 8)
        pltpu.make_async_copy(x_hbm.at[pl.ds(s, CHUNK)], rb, sem).wait()
        pltpu.make_async_copy(b_hbm.at[pl.ds(s, CHUNK)],
                              ib.at[pl.ds(0, CHUNK)], sem).wait()

    def _body(i, rb, ib):
        seg = ib[pl.ds(i, 16)][0]
        for k in range(D_X // 16):
            v = rb[i, pl.ds(16 * k, 16)]
            plsc.addupdate(acc.at[seg, pl.ds(16 * k, 16)], v)
        plsc.addupdate(accc.at[pl.ds(seg * 16, 16)],
                       jnp.ones((16,), jnp.float32))

    def _process(rb, ib):
        @plsc.parallel_loop(0, CHUNK, 1, unroll=16)
        def _row(i):
            _body(i, rb, ib)

    _start(0, rb0, ib0, sem0)

    @pl.loop(0, NFULL, step=2)
    def _chunk(c):
        _start(c + 1, rb1, ib1, sem1)
        _wait(c, rb0, ib0, sem0)
        _process(rb0, ib0)

        @pl.when(c + 2 < NFULL)
        def _():
            _start(c + 2, rb0, ib0, sem0)

        _wait(c + 1, rb1, ib1, sem1)
        _process(rb1, ib1)

    s_t = pl.multiple_of(row0 + NFULL * CHUNK, 8)
    rows_t = rem_o * 8

    @pl.when(rem_o == 6)
    def _tail6():
        pltpu.sync_copy(x_hbm.at[pl.ds(s_t, 48)], rb0.at[pl.ds(0, 48)])
        pltpu.sync_copy(b_hbm.at[pl.ds(s_t, 48)], ib0.at[pl.ds(0, 48)])

    @pl.when(rem_o == 7)
    def _tail7():
        pltpu.sync_copy(x_hbm.at[pl.ds(s_t, 56)], rb0.at[pl.ds(0, 56)])
        pltpu.sync_copy(b_hbm.at[pl.ds(s_t, 56)], ib0.at[pl.ds(0, 56)])

    @pl.loop(0, rows_t)
    def _tailrow(i):
        _body(i, rb0, ib0)

    pltpu.sync_copy(acc, out_hbm.at[wid])
    pltpu.sync_copy(accc, outc_hbm.at[pl.ds(wid * G * 16, G * 16)])


def _sc_segsum(x, b32):
    mesh = plsc.VectorSubcoreMesh(core_axis_name="c", subcore_axis_name="s")
    return pl.kernel(
        _sc_segsum_body,
        out_type=[
            jax.ShapeDtypeStruct((NW, G, D_X), jnp.float32),
            jax.ShapeDtypeStruct((NW * G * 16,), jnp.float32),
        ],
        mesh=mesh,
        scratch_types=[
            pltpu.VMEM((CHUNK, D_X), jnp.float32),
            pltpu.VMEM((CHUNK, D_X), jnp.float32),
            pltpu.VMEM((CHUNK + 16,), jnp.int32),
            pltpu.VMEM((CHUNK + 16,), jnp.int32),
            pltpu.VMEM((G, D_X), jnp.float32),
            pltpu.VMEM((G * 16,), jnp.float32),
            pltpu.SemaphoreType.DMA,
            pltpu.SemaphoreType.DMA,
        ],
    )(x, b32)


def _tc_pe_body(peT_ref, b_ref, accp):
    seg = lax.broadcasted_iota(jnp.int32, (1, G), 1)
    acc = jnp.zeros((PE_DIM, G), jnp.float32)
    for j in range(NB):
        ids = b_ref[0, pl.ds(j * BLK, BLK)]
        onehot = (ids[:, None] == seg).astype(jnp.float32)
        peb = peT_ref[:, pl.ds(j * BLK, BLK)]
        acc += lax.dot_general(
            peb, onehot, (((1,), (0,)), ((), ())),
            preferred_element_type=jnp.float32)
    accp[...] = acc


def _tc_pe(peT, b32):
    return pl.pallas_call(
        _tc_pe_body,
        out_shape=jax.ShapeDtypeStruct((PE_DIM, G), jnp.float32),
    )(peT, b32.reshape(1, N))


def _tc_combine_body(parts, partsc, accp8, W_enc, b_enc, W1, b1, W2, b2,
                     out_ref):
    psum = jnp.sum(parts[...], axis=0)
    cnt = jnp.maximum(jnp.sum(partsc[...], axis=0)[:, 0:1], 1.0)
    pre = (lax.dot_general(psum, W_enc[:, :D_X],
                           (((1,), (1,)), ((), ())),
                           preferred_element_type=jnp.float32)
           + lax.dot_general(accp8[...], W_enc[:, D_X:],
                             (((0,), (1,)), ((), ())),
                             preferred_element_type=jnp.float32))
    h = pre / cnt + b_enc[...]
    h1 = jnp.maximum(
        lax.dot_general(h, W1[...], (((1,), (1,)), ((), ())),
                        preferred_element_type=jnp.float32) + b1[...], 0.0)
    out_ref[...] = (
        lax.dot_general(h1, W2[...], (((1,), (1,)), ((), ())),
                        preferred_element_type=jnp.float32) + b2[...])


def _tc_combine(parts, partsc, accp8, W_enc, b_enc, W1, b1, W2, b2):
    return pl.pallas_call(
        _tc_combine_body,
        out_shape=jax.ShapeDtypeStruct((G, OUT), jnp.float32),
    )(parts, partsc, accp8, W_enc, b_enc.reshape(1, HID), W1,
      b1.reshape(1, HID), W2, b2.reshape(1, OUT))


def kernel(x, pe, batch, W_enc, b_enc, W1, b1, W2, b2):
    b32 = batch.astype(jnp.int32)
    parts, partsc = _sc_segsum(x, b32)
    accp8 = _tc_pe(pe.T, b32)
    return _tc_combine(parts, partsc.reshape(NW, G, 16), accp8,
                       W_enc, b_enc, W1, b1, W2, b2)


# speedup vs baseline: 1.2810x; 1.2810x over previous
import jax
import jax.numpy as jnp
from jax import lax
from jax.experimental import pallas as pl
from jax.experimental.pallas import tpu as pltpu
from jax.experimental.pallas import tpu_sc as plsc

N = 100000
D_X = 128
PE_DIM = 8
HID = 128
OUT = 16
G = 512

BLK = 2000
NB = N // BLK

N_SC = 76000
NB_TC0 = N_SC // BLK

NW = 32
NOCT = N_SC // 8
BASE_O = NOCT // NW
EXTRA_O = NOCT % NW
CHUNK = 128
NFULL = 18
REM_LO = BASE_O - NFULL * (CHUNK // 8)
REM_HI = REM_LO + 1


def _sc_segsum_body(x_hbm, b_hbm, out_hbm, rb0, rb1, ib0, ib1, acc,
                    sem0, sem1):
    wid = lax.axis_index("c") * 16 + lax.axis_index("s")
    start_o = wid * BASE_O + jnp.minimum(wid, EXTRA_O)
    count_o = BASE_O + (wid < EXTRA_O).astype(jnp.int32)
    row0 = pl.multiple_of(start_o * 8, 8)
    rem_o = count_o - NFULL * (CHUNK // 8)

    @pl.loop(0, G)
    def _zero(r):
        for k in range(D_X // 16):
            acc[r, pl.ds(16 * k, 16)] = jnp.zeros((16,), jnp.float32)

    def _start(c, rb, ib, sem):
        s = pl.multiple_of(row0 + c * CHUNK, 8)
        pltpu.async_copy(x_hbm.at[pl.ds(s, CHUNK)], rb, sem)
        pltpu.async_copy(b_hbm.at[pl.ds(s, CHUNK)],
                         ib.at[pl.ds(0, CHUNK)], sem)

    def _wait(c, rb, ib, sem):
        s = pl.multiple_of(row0 + c * CHUNK, 8)
        pltpu.make_async_copy(x_hbm.at[pl.ds(s, CHUNK)], rb, sem).wait()
        pltpu.make_async_copy(b_hbm.at[pl.ds(s, CHUNK)],
                              ib.at[pl.ds(0, CHUNK)], sem).wait()

    def _body(i, rb, ib):
        seg = ib[pl.ds(i, 16)][0]
        for k in range(D_X // 16):
            v = rb[i, pl.ds(16 * k, 16)]
            plsc.addupdate(acc.at[seg, pl.ds(16 * k, 16)], v)

    def _process(rb, ib):
        @plsc.parallel_loop(0, CHUNK, 1, unroll=8)
        def _row(i):
            _body(i, rb, ib)

    _start(0, rb0, ib0, sem0)

    @pl.loop(0, NFULL, step=2)
    def _chunk(c):
        _start(c + 1, rb1, ib1, sem1)
        _wait(c, rb0, ib0, sem0)
        _process(rb0, ib0)

        @pl.when(c + 2 < NFULL)
        def _():
            _start(c + 2, rb0, ib0, sem0)

        _wait(c + 1, rb1, ib1, sem1)
        _process(rb1, ib1)

    s_t = pl.multiple_of(row0 + NFULL * CHUNK, 8)
    rows_t = rem_o * 8

    @pl.when(rem_o == REM_LO)
    def _tail_lo():
        pltpu.sync_copy(x_hbm.at[pl.ds(s_t, REM_LO * 8)],
                        rb0.at[pl.ds(0, REM_LO * 8)])
        pltpu.sync_copy(b_hbm.at[pl.ds(s_t, REM_LO * 8)],
                        ib0.at[pl.ds(0, REM_LO * 8)])

    @pl.when(rem_o == REM_HI)
    def _tail_hi():
        pltpu.sync_copy(x_hbm.at[pl.ds(s_t, REM_HI * 8)],
                        rb0.at[pl.ds(0, REM_HI * 8)])
        pltpu.sync_copy(b_hbm.at[pl.ds(s_t, REM_HI * 8)],
                        ib0.at[pl.ds(0, REM_HI * 8)])

    @pl.loop(0, rows_t)
    def _tailrow(i):
        _body(i, rb0, ib0)

    pltpu.sync_copy(acc, out_hbm.at[wid])


def _sc_segsum(x, b32):
    mesh = plsc.VectorSubcoreMesh(core_axis_name="c", subcore_axis_name="s")
    return pl.kernel(
        _sc_segsum_body,
        out_type=jax.ShapeDtypeStruct((NW, G, D_X), jnp.float32),
        mesh=mesh,
        scratch_types=[
            pltpu.VMEM((CHUNK, D_X), jnp.float32),
            pltpu.VMEM((CHUNK, D_X), jnp.float32),
            pltpu.VMEM((CHUNK + 16,), jnp.int32),
            pltpu.VMEM((CHUNK + 16,), jnp.int32),
            pltpu.VMEM((G, D_X), jnp.float32),
            pltpu.SemaphoreType.DMA,
            pltpu.SemaphoreType.DMA,
        ],
    )(x, b32)


def _tc_pe_body(peT_ref, b_ref, accp):
    seg = lax.broadcasted_iota(jnp.int32, (1, G), 1)
    acc = jnp.zeros((PE_DIM + 1, G), jnp.float32)
    for j in range(NB):
        ids = b_ref[0, pl.ds(j * BLK, BLK)]
        onehot = (ids[:, None] == seg).astype(jnp.float32)
        peb = peT_ref[:, pl.ds(j * BLK, BLK)]
        pe1 = jnp.concatenate(
            [peb, jnp.ones((1, BLK), jnp.float32)], axis=0)
        acc += lax.dot_general(
            pe1, onehot, (((1,), (0,)), ((), ())),
            preferred_element_type=jnp.float32)
    accp[...] = acc


def _tc_pe(peT, b32):
    return pl.pallas_call(
        _tc_pe_body,
        out_shape=jax.ShapeDtypeStruct((PE_DIM + 1, G), jnp.float32),
    )(peT, b32.reshape(1, N))


def _tc_x_body(xb, bb, accx):
    step = pl.program_id(0)

    @pl.when(step == 0)
    def _init():
        accx[...] = jnp.zeros_like(accx)

    ids = bb[0, 0, :]
    seg = lax.broadcasted_iota(jnp.int32, (1, G), 1)
    onehot = (ids[:, None] == seg).astype(jnp.float32)
    accx[...] += lax.dot_general(
        onehot, xb[...], (((0,), (0,)), ((), ())),
        preferred_element_type=jnp.float32)


def _tc_x(x, batch3):
    return pl.pallas_call(
        _tc_x_body,
        grid=(NB - NB_TC0,),
        in_specs=[
            pl.BlockSpec((BLK, D_X), lambda i: (i + NB_TC0, 0)),
            pl.BlockSpec((1, 1, BLK), lambda i: (i + NB_TC0, 0, 0)),
        ],
        out_specs=pl.BlockSpec((G, D_X), lambda i: (0, 0)),
        out_shape=jax.ShapeDtypeStruct((G, D_X), jnp.float32),
    )(x, batch3)


def _tc_combine_body(parts, accx2, accp9, W_enc, b_enc, W1, b1, W2, b2,
                     out_ref):
    psum = jnp.sum(parts[...], axis=0) + accx2[...]
    gi = lax.broadcasted_iota(jnp.int32, (G, G), 0)
    gj = lax.broadcasted_iota(jnp.int32, (G, G), 1)
    ident = (gi == gj).astype(jnp.float32)
    cnt_col = lax.dot_general(
        ident, accp9[PE_DIM:PE_DIM + 1, :],
        (((1,), (1,)), ((), ())),
        preferred_element_type=jnp.float32)
    cnt = jnp.maximum(cnt_col, 1.0)
    pre = (lax.dot_general(psum, W_enc[:, :D_X],
                           (((1,), (1,)), ((), ())),
                           preferred_element_type=jnp.float32)
           + lax.dot_general(accp9[:PE_DIM, :], W_enc[:, D_X:],
                             (((0,), (1,)), ((), ())),
                             preferred_element_type=jnp.float32))
    h = pre / cnt + b_enc[...]
    h1 = jnp.maximum(
        lax.dot_general(h, W1[...], (((1,), (1,)), ((), ())),
                        preferred_element_type=jnp.float32) + b1[...], 0.0)
    out_ref[...] = (
        lax.dot_general(h1, W2[...], (((1,), (1,)), ((), ())),
                        preferred_element_type=jnp.float32) + b2[...])


def _tc_combine(parts, accx2, accp9, W_enc, b_enc, W1, b1, W2, b2):
    return pl.pallas_call(
        _tc_combine_body,
        out_shape=jax.ShapeDtypeStruct((G, OUT), jnp.float32),
    )(parts, accx2, accp9, W_enc, b_enc.reshape(1, HID), W1,
      b1.reshape(1, HID), W2, b2.reshape(1, OUT))


def kernel(x, pe, batch, W_enc, b_enc, W1, b1, W2, b2):
    b32 = batch.astype(jnp.int32)
    parts = _sc_segsum(x, b32)
    accp9 = _tc_pe(pe.T, b32)
    accx2 = _tc_x(x, b32.reshape(NB, 1, BLK))
    return _tc_combine(parts, accx2, accp9, W_enc, b_enc, W1, b1, W2, b2)
